# Initial kernel scaffold; baseline (speedup 1.0000x reference)
import jax
import jax.numpy as jnp
from jax.experimental import pallas as pl


def kernel(embedding_matrix, mask):
    raise NotImplementedError("write your pallas kernel here")


# speedup vs baseline: 1.0082x; 1.0082x over previous
import jax
import jax.numpy as jnp
from jax.experimental import pallas as pl
from jax.experimental.pallas import tpu as pltpu


def _pool_kernel(x_ref, m_ref, o_ref):
    s = jnp.sum(x_ref[...], axis=1, dtype=jnp.float32)
    cnt = jnp.sum(m_ref[...].astype(jnp.float32), axis=-1,
                  keepdims=True)
    o_ref[...] = s / (cnt + 1e-12)


def _pick_tb(B, L, D, itemsize):
    slab_limit = 8 * 1024 * 1024
    row = L * D * itemsize
    tb = max(8, min(B, slab_limit // max(row, 1) // 8 * 8))
    while tb > 8 and B % tb != 0:
        tb -= 8
    return tb


def kernel(embedding_matrix, mask):
    x = embedding_matrix
    B, L, D = x.shape
    tb = _pick_tb(B, L, D, x.dtype.itemsize)
    grid = (pl.cdiv(B, tb),)
    return pl.pallas_call(
        _pool_kernel,
        out_shape=jax.ShapeDtypeStruct((B, D), jnp.float32),
        grid=grid,
        in_specs=[
            pl.BlockSpec((tb, L, D), lambda b: (b, 0, 0)),
            pl.BlockSpec((tb, L), lambda b: (b, 0)),
        ],
        out_specs=pl.BlockSpec((tb, D), lambda b: (b, 0)),
        compiler_params=pltpu.CompilerParams(
            dimension_semantics=("parallel",),
            vmem_limit_bytes=48 * 1024 * 1024,
        ),
    )(x, mask)
